# baseline (speedup 1.0000x reference)
import functools

import jax
import jax.numpy as jnp
from jax import lax
from jax.experimental import pallas as pl
from jax.experimental.pallas import tpu as pltpu
from jax.experimental.pallas import tpu_sc as plsc

_ORI_V = 100000
_THINK_V = 100000
_EMBED = 64
_ROWS = 4096
_COLS = 200
_NC = 2
_NS = 16
_NW = _NC * _NS
_SLAB = _ROWS // _NW
_LANES = 16
_CTILES = _EMBED // 8
_TWORDS = _CTILES * 8 * _SLAB
_BUF = 160
_BTOP = 144
_DEPTH = 4
_TILE_B = _LANES * _EMBED * 4


def _make_kernel():
    mesh = plsc.VectorSubcoreMesh(core_axis_name="c", subcore_axis_name="s")

    @functools.partial(
        pl.kernel,
        mesh=mesh,
        out_type=jax.ShapeDtypeStruct((_COLS, _CTILES, _NW, 8 * _SLAB),
                                      jnp.float32),
        scratch_types=[
            pltpu.VMEM((_COLS, _SLAB), jnp.int32),
            pltpu.VMEM((_DEPTH, _BUF, _EMBED), jnp.float32),
            pltpu.VMEM((_DEPTH, _BUF), jnp.int32),
            pltpu.VMEM((_DEPTH, _BUF), jnp.int32),
            pltpu.VMEM((_DEPTH, _SLAB), jnp.int32),
            pltpu.VMEM((2, _TWORDS), jnp.float32),
            pltpu.SMEM((_DEPTH, 2), jnp.int32),
            pltpu.SemaphoreType.DMA,
            pltpu.SemaphoreType.DMA,
            pltpu.SemaphoreType.DMA,
            pltpu.SemaphoreType.DMA,
            pltpu.SemaphoreType.DMA,
            pltpu.SemaphoreType.DMA,
            pltpu.SemaphoreType.DMA,
        ],
        compiler_params=pltpu.CompilerParams(
            needs_layout_passes=False, use_tc_tiling_on_sc=False),
    )
    def combined(ids_hbm, ori_hbm, think_hbm, out_hbm,
                 idx_all, rows_v, list_a, list_b, inv_v, trans_v, cnt_s,
                 isem, gsem0, gsem1, gsem2, gsem3, wsem0, wsem1):
        gsems = (gsem0, gsem1, gsem2, gsem3)
        wsems = (wsem0, wsem1)
        wid = lax.axis_index("s") * _NC + lax.axis_index("c")
        iota = lax.iota(jnp.int32, _LANES)
        us = [(iota + k) % _LANES for k in range(_LANES)]
        dvs = [(us[k] // 8) * (8 * _SLAB) + (us[k] % 8) * _SLAB + iota
               for k in range(_LANES)]

        pltpu.async_copy(
            ids_hbm.at[:, pl.ds(wid * _SLAB, _SLAB)], idx_all, isem).wait()

        zeros = jnp.zeros((_LANES,), jnp.int32)
        for p in range(_DEPTH):
            for o in range(_BUF // _LANES):
                list_a[p, pl.ds(o * _LANES, _LANES)] = zeros
                list_b[p, pl.ds(o * _LANES, _LANES)] = zeros

        def fire_gather(j, p):
            ids_row = idx_all.at[j]
            la = list_a.at[p]
            lb = list_b.at[p]
            inv = inv_v.at[p]
            ca = jnp.int32(0)
            cb = jnp.int32(0)
            for b in range(_SLAB // _LANES):
                v = ids_row[pl.ds(b * _LANES, _LANES)]
                m = v < _ORI_V
                pca = plsc.cumsum(m.astype(jnp.int32)) - 1
                pcb = plsc.cumsum(jnp.logical_not(m).astype(jnp.int32)) - 1
                inv[pl.ds(b * _LANES, _LANES)] = jnp.where(
                    m, ca + pca, (_BTOP - 1) - cb - pcb)
                plsc.store_compressed(la.at[pl.ds(ca, _LANES)], v, mask=m)
                rv = lax.rev(v, (0,))
                rm = rv >= _ORI_V
                na = jnp.sum(m.astype(jnp.int32))
                nb = _LANES - na
                plsc.store_compressed(
                    lb.at[pl.ds(_BTOP - cb - nb, _LANES)],
                    rv - _ORI_V, mask=rm)
                ca = ca + na
                cb = cb + nb
            cnt_s[p, 0] = ca
            cnt_s[p, 1] = cb

            nta = (ca + _LANES - 1) // _LANES

            def fa(t, c2):
                s = t * _LANES
                pltpu.async_copy(
                    ori_hbm.at[la.at[pl.ds(s, _LANES)]],
                    rows_v.at[p].at[pl.ds(s, _LANES)], gsems[p])
                return c2

            lax.fori_loop(0, nta, fa, 0)

            ntb = (cb + _LANES - 1) // _LANES

            def fb(t, c2):
                s = (_BTOP - _LANES) - t * _LANES
                pltpu.async_copy(
                    think_hbm.at[lb.at[pl.ds(s, _LANES)]],
                    rows_v.at[p].at[pl.ds(s, _LANES)], gsems[p])
                return c2

            lax.fori_loop(0, ntb, fb, 0)

        def wait_gather(j, p):
            ca = cnt_s[p, 0]
            cb = cnt_s[p, 1]
            nt = (ca + _LANES - 1) // _LANES + (cb + _LANES - 1) // _LANES

            def w(t, c2):
                pltpu.make_async_copy(
                    ori_hbm.at[list_a.at[p].at[pl.ds(0, _LANES)]],
                    rows_v.at[p].at[pl.ds(0, _LANES)], gsems[p]).wait()
                return c2

            lax.fori_loop(0, nt, w, 0)

        def fire_write(j, p):
            for tc in range(_CTILES):
                pltpu.async_copy(
                    trans_v.at[p, pl.ds(tc * 8 * _SLAB, 8 * _SLAB)],
                    out_hbm.at[j, tc, wid, :], wsems[p])

        def wait_write(j, p):
            for tc in range(_CTILES):
                pltpu.make_async_copy(
                    trans_v.at[p, pl.ds(tc * 8 * _SLAB, 8 * _SLAB)],
                    out_hbm.at[j, tc, wid, :], wsems[p]).wait()

        def transpose(pg, pw):
            rows = rows_v.at[pg]
            inv = inv_v.at[pg]
            dst = trans_v.at[pw]

            @plsc.parallel_loop(0, 32, unroll=4)
            def _(t):
                ib = t // 4
                cb = t % 4
                iv = inv[pl.ds(ib * _LANES, _LANES)]
                cbase = cb * _LANES
                obase = cb * 2 * (8 * _SLAB) + ib * _LANES
                for k in range(_LANES):
                    vals = plsc.load_gather(rows, [iv, us[k] + cbase])
                    plsc.store_scatter(dst, [dvs[k] + obase], vals)

        fire_gather(0, 0)
        fire_gather(1, 1)
        fire_gather(2, 2)

        def body(gp, carry):
            for sub in range(4):
                j = 4 * gp + sub
                pg = sub
                pw = sub % 2

                @pl.when(j + 3 < _COLS)
                def _():
                    fire_gather(j + 3, (sub + 3) % 4)

                wait_gather(j, pg)

                @pl.when(j >= 2)
                def _():
                    wait_write(j - 2, pw)

                transpose(pg, pw)
                fire_write(j, pw)
            return carry

        lax.fori_loop(0, _COLS // 4, body, 0)
        wait_write(_COLS - 2, 0)
        wait_write(_COLS - 1, 1)

    return combined


_COMBINED = _make_kernel()


def kernel(ids, ori_weight, think_weight):
    out5 = _COMBINED(ids.astype(jnp.int32).T, ori_weight, think_weight)
    out = out5.reshape(_COLS, _CTILES, _NW, 8, _SLAB)
    out = out.transpose(2, 4, 0, 1, 3)
    return out.reshape(_ROWS, _COLS, _EMBED)


# speedup vs baseline: 2.0721x; 2.0721x over previous
import functools

import jax
import jax.numpy as jnp
from jax import lax
from jax.experimental import pallas as pl
from jax.experimental.pallas import tpu as pltpu
from jax.experimental.pallas import tpu_sc as plsc

_ORI_V = 100000
_THINK_V = 100000
_EMBED = 64
_ROWS = 4096
_COLS = 200
_NC = 2
_NS = 16
_NW = _NC * _NS
_SLAB = _ROWS // _NW
_LANES = 16
_CTILES = _EMBED // 8
_TWORDS = _CTILES * 8 * _SLAB


def _make_kernel():
    mesh = plsc.VectorSubcoreMesh(core_axis_name="c", subcore_axis_name="s")

    @functools.partial(
        pl.kernel,
        mesh=mesh,
        out_type=jax.ShapeDtypeStruct((_COLS, _CTILES, _NW, 8 * _SLAB),
                                      jnp.float32),
        scratch_types=[
            pltpu.VMEM((_COLS, _SLAB), jnp.int32),
            pltpu.VMEM((4, _SLAB, _EMBED), jnp.float32),
            pltpu.VMEM((2, _TWORDS), jnp.float32),
            pltpu.SemaphoreType.DMA,
            pltpu.SemaphoreType.DMA,
            pltpu.SemaphoreType.DMA,
            pltpu.SemaphoreType.DMA,
            pltpu.SemaphoreType.DMA,
            pltpu.SemaphoreType.DMA,
            pltpu.SemaphoreType.DMA,
        ],
        compiler_params=pltpu.CompilerParams(
            needs_layout_passes=False, use_tc_tiling_on_sc=False),
    )
    def combined(ids_hbm, comb_hbm, out_hbm,
                 idx_all, rows_v, trans_v, isem,
                 gsem0, gsem1, gsem2, gsem3, wsem0, wsem1):
        gsems = (gsem0, gsem1, gsem2, gsem3)
        wsems = (wsem0, wsem1)
        wid = lax.axis_index("s") * _NC + lax.axis_index("c")
        iota = lax.iota(jnp.int32, _LANES)
        us = [(iota + k) % _LANES for k in range(_LANES)]
        cvs = [us[k] for k in range(_LANES)]
        dvs = [(us[k] // 8) * (8 * _SLAB) + (us[k] % 8) * _SLAB + iota
               for k in range(_LANES)]

        pltpu.async_copy(
            ids_hbm.at[:, pl.ds(wid * _SLAB, _SLAB)], idx_all, isem).wait()

        def fire_gather(j, p):
            pltpu.async_copy(
                comb_hbm.at[idx_all.at[j]], rows_v.at[p], gsems[p])

        def wait_gather(j, p):
            pltpu.make_async_copy(
                comb_hbm.at[idx_all.at[j]], rows_v.at[p], gsems[p]).wait()

        def fire_write(j, p):
            for tc in range(_CTILES):
                pltpu.async_copy(
                    trans_v.at[p, pl.ds(tc * 8 * _SLAB, 8 * _SLAB)],
                    out_hbm.at[j, tc, wid, :], wsems[p])

        def wait_write(j, p):
            for tc in range(_CTILES):
                pltpu.make_async_copy(
                    trans_v.at[p, pl.ds(tc * 8 * _SLAB, 8 * _SLAB)],
                    out_hbm.at[j, tc, wid, :], wsems[p]).wait()

        def transpose(pg, pw):
            rows = rows_v.at[pg]
            dst = trans_v.at[pw]

            @plsc.parallel_loop(0, 32, unroll=4)
            def _(t):
                ib = t // 4
                cb = t % 4
                iv = ib * _LANES + iota
                cbase = cb * _LANES
                obase = cb * 2 * (8 * _SLAB) + ib * _LANES
                for k in range(_LANES):
                    vals = plsc.load_gather(rows, [iv, cvs[k] + cbase])
                    plsc.store_scatter(dst, [dvs[k] + obase], vals)

        fire_gather(0, 0)
        fire_gather(1, 1)
        fire_gather(2, 2)

        def body(gp, carry):
            for sub in range(4):
                j = 4 * gp + sub
                pg = sub
                pw = sub % 2

                @pl.when(j + 3 < _COLS)
                def _():
                    fire_gather(j + 3, (sub + 3) % 4)

                wait_gather(j, pg)

                @pl.when(j >= 2)
                def _():
                    wait_write(j - 2, pw)

                transpose(pg, pw)
                fire_write(j, pw)
            return carry

        lax.fori_loop(0, _COLS // 4, body, 0)
        wait_write(_COLS - 2, 0)
        wait_write(_COLS - 1, 1)

    return combined


_COMBINED = _make_kernel()


def kernel(ids, ori_weight, think_weight):
    comb = jnp.concatenate([ori_weight, think_weight], axis=0)
    out5 = _COMBINED(ids.astype(jnp.int32).T, comb)
    out = out5.reshape(_COLS, _CTILES, _NW, 8, _SLAB)
    out = out.transpose(2, 4, 0, 1, 3)
    return out.reshape(_ROWS, _COLS, _EMBED)
